# Initial kernel scaffold; baseline (speedup 1.0000x reference)
import jax
import jax.numpy as jnp
from jax.experimental import pallas as pl


def kernel(node_features, edge_indices, edge_features, W_in, b_in, msg_w1, msg_b1, msg_w2, msg_b2, upd_w1, upd_b1, upd_w2, upd_b2, W_out, b_out):
    raise NotImplementedError("write your pallas kernel here")


# speedup vs baseline: 7.3720x; 7.3720x over previous
import functools

import jax
import jax.numpy as jnp
from jax import lax
from jax.experimental import pallas as pl
from jax.experimental.pallas import tpu as pltpu
from jax.experimental.pallas import tpu_sc as plsc

N = 50000
E = 800000
ND = 128
HD = 64
ED = 16

NC = 2
NS = 16
CG = 128
E_PAD = 32 * 196 * CG
EW = E_PAD // (NC * NS)
ET = E_PAD // NS
NROWS_SP = 50016

_SC_MESH = plsc.VectorSubcoreMesh(core_axis_name="c", subcore_axis_name="s")
_SC_PARAMS = pltpu.CompilerParams(use_tc_tiling_on_sc=False)


def _mm_bias_body(x_ref, w_ref, b_ref, o_ref):
    o_ref[...] = jnp.dot(x_ref[...], w_ref[...],
                         preferred_element_type=jnp.float32) + b_ref[...]


def _tc_mm_bias(x, w, b, br):
    r, k = x.shape
    c = w.shape[1]
    return pl.pallas_call(
        _mm_bias_body,
        grid=(r // br,),
        in_specs=[pl.BlockSpec((br, k), lambda i: (i, 0)),
                  pl.BlockSpec((k, c), lambda i: (0, 0)),
                  pl.BlockSpec((1, c), lambda i: (0, 0))],
        out_specs=pl.BlockSpec((br, c), lambda i: (i, 0)),
        out_shape=jax.ShapeDtypeStruct((r, c), jnp.float32),
    )(x, w, b.reshape(1, c))


def _tables_body(h_ref, w_ref, o_ref):
    h = h_ref[...]
    o_ref[0] = jnp.dot(h, w_ref[0:HD], preferred_element_type=jnp.float32)
    o_ref[1] = jnp.dot(h, w_ref[HD:2 * HD], preferred_element_type=jnp.float32)


def _tc_tables(h, w12, br):
    return pl.pallas_call(
        _tables_body,
        grid=(N // br,),
        in_specs=[pl.BlockSpec((br, HD), lambda i: (i, 0)),
                  pl.BlockSpec((2 * HD, HD), lambda i: (0, 0))],
        out_specs=pl.BlockSpec((2, br, HD), lambda i: (0, i, 0)),
        out_shape=jax.ShapeDtypeStruct((2, N, HD), jnp.float32),
    )(h, w12)


def _edge_mlp_body(pre_ref, ef_ref, wc_ref, b1_ref, w2_ref, b2_ref, o_ref):
    mi = pre_ref[...] + jnp.dot(ef_ref[...], wc_ref[...],
                                preferred_element_type=jnp.float32) + b1_ref[...]
    o_ref[...] = jnp.dot(jax.nn.gelu(mi), w2_ref[...],
                         preferred_element_type=jnp.float32) + b2_ref[...]


def _tc_edge_mlp(pre, ef, wc, b1, w2, b2, be):
    return pl.pallas_call(
        _edge_mlp_body,
        grid=(E_PAD // be,),
        in_specs=[pl.BlockSpec((be, HD), lambda i: (i, 0)),
                  pl.BlockSpec((be, ED), lambda i: (i, 0)),
                  pl.BlockSpec((ED, HD), lambda i: (0, 0)),
                  pl.BlockSpec((1, HD), lambda i: (0, 0)),
                  pl.BlockSpec((HD, HD), lambda i: (0, 0)),
                  pl.BlockSpec((1, HD), lambda i: (0, 0))],
        out_specs=pl.BlockSpec((be, HD), lambda i: (i, 0)),
        out_shape=jax.ShapeDtypeStruct((E_PAD, HD), jnp.float32),
    )(pre, ef, wc, b1.reshape(1, HD), w2, b2.reshape(1, HD))


def _update_body(h_ref, a_ref, w1_ref, b1_ref, w2_ref, b2_ref, o_ref):
    h = h_ref[...]
    ui = (jnp.dot(h, w1_ref[0:HD], preferred_element_type=jnp.float32)
          + jnp.dot(a_ref[...], w1_ref[HD:2 * HD],
                    preferred_element_type=jnp.float32)
          + b1_ref[...])
    o_ref[...] = h + jnp.dot(jax.nn.gelu(ui), w2_ref[...],
                             preferred_element_type=jnp.float32) + b2_ref[...]


def _tc_update(h, agg, w1, b1, w2, b2, br):
    return pl.pallas_call(
        _update_body,
        grid=(N // br,),
        in_specs=[pl.BlockSpec((br, HD), lambda i: (i, 0)),
                  pl.BlockSpec((br, HD), lambda i: (i, 0)),
                  pl.BlockSpec((2 * HD, HD), lambda i: (0, 0)),
                  pl.BlockSpec((1, HD), lambda i: (0, 0)),
                  pl.BlockSpec((HD, HD), lambda i: (0, 0)),
                  pl.BlockSpec((1, HD), lambda i: (0, 0))],
        out_specs=pl.BlockSpec((br, HD), lambda i: (i, 0)),
        out_shape=jax.ShapeDtypeStruct((N, HD), jnp.float32),
    )(h, agg, w1, b1.reshape(1, HD), w2, b2.reshape(1, HD))


def _gather_body(ts_ref, td_ref, src_ref, dst_ref, pre_ref,
                 idx_s, idx_d, buf_a, buf_b, sem_a, sem_b):
    cid = lax.axis_index("c")
    sid = lax.axis_index("s")
    base = (sid * NC + cid) * EW

    def chunk(g, carry):
        e0 = base + g * CG
        pltpu.sync_copy(src_ref.at[pl.ds(e0, CG)], idx_s)
        pltpu.sync_copy(dst_ref.at[pl.ds(e0, CG)], idx_d)
        cp_a = pltpu.async_copy(ts_ref.at[idx_s], buf_a, sem_a)
        cp_b = pltpu.async_copy(td_ref.at[idx_d], buf_b, sem_b)
        cp_a.wait()
        cp_b.wait()

        def addrow(i, c2):
            for c4 in range(HD // 16):
                sl = pl.ds(c4 * 16, 16)
                buf_a[i, sl] = buf_a[i, sl] + buf_b[i, sl]
            return c2

        lax.fori_loop(0, CG, addrow, 0)
        pltpu.sync_copy(buf_a, pre_ref.at[pl.ds(e0, CG)])
        return carry

    lax.fori_loop(0, EW // CG, chunk, 0)


_sc_gather = pl.kernel(
    _gather_body,
    out_type=jax.ShapeDtypeStruct((E_PAD, HD), jnp.float32),
    mesh=_SC_MESH,
    scratch_types=[
        pltpu.VMEM((CG,), jnp.int32),
        pltpu.VMEM((CG,), jnp.int32),
        pltpu.VMEM((CG, HD), jnp.float32),
        pltpu.VMEM((CG, HD), jnp.float32),
        pltpu.SemaphoreType.DMA,
        pltpu.SemaphoreType.DMA,
    ],
    compiler_params=_SC_PARAMS,
)


def _scatter_body(m_ref, dst_ref, agg_ref, idxb, mb, ob, aggs):
    cid = lax.axis_index("c")
    sid = lax.axis_index("s")
    col0 = cid * (HD // NC)

    def zrow(i, carry):
        ob[i, pl.ds(0, 16)] = jnp.zeros((16,), jnp.float32)
        ob[i, pl.ds(16, 16)] = jnp.zeros((16,), jnp.float32)
        return carry

    lax.fori_loop(0, 125, zrow, 0)

    def zcopy(j, carry):
        pltpu.sync_copy(ob, aggs.at[pl.ds(sid * 3126 + j * 125, 125)])
        return carry

    lax.fori_loop(0, 25, zcopy, 0)
    pltpu.sync_copy(ob.at[pl.ds(0, 1)], aggs.at[pl.ds(sid * 3126 + 3125, 1)])
    plsc.subcore_barrier()

    base = sid * ET

    def chunk(g, carry):
        e0 = base + g * CG
        pltpu.sync_copy(dst_ref.at[pl.ds(e0, CG)], idxb)
        pltpu.sync_copy(m_ref.at[pl.ds(e0, CG), pl.ds(col0, HD // NC)], mb)
        pltpu.sync_copy(mb, aggs.at[idxb], add=True)
        return carry

    lax.fori_loop(0, ET // CG, chunk, 0)
    plsc.subcore_barrier()

    def wout(k, carry):
        r0 = sid * 3125 + k * 125
        pltpu.sync_copy(aggs.at[pl.ds(r0, 125)], ob)
        pltpu.sync_copy(ob, agg_ref.at[pl.ds(r0, 125), pl.ds(col0, HD // NC)])
        return carry

    lax.fori_loop(0, 25, wout, 0)


_sc_scatter = pl.kernel(
    _scatter_body,
    out_type=jax.ShapeDtypeStruct((N, HD), jnp.float32),
    mesh=_SC_MESH,
    scratch_types=[
        pltpu.VMEM((CG,), jnp.int32),
        pltpu.VMEM((CG, HD // NC), jnp.float32),
        pltpu.VMEM((125, HD // NC), jnp.float32),
        pltpu.VMEM_SHARED((NROWS_SP, HD // NC), jnp.float32),
    ],
    compiler_params=_SC_PARAMS,
)


def kernel(node_features, edge_indices, edge_features, W_in, b_in,
           msg_w1, msg_b1, msg_w2, msg_b2,
           upd_w1, upd_b1, upd_w2, upd_b2, W_out, b_out):
    nf = node_features[0]
    src = edge_indices[0, :, 0].astype(jnp.int32)
    dst = edge_indices[0, :, 1].astype(jnp.int32)
    ef = edge_features[0]

    pad = E_PAD - E
    zpad_i = jnp.zeros((pad,), jnp.int32)
    src_g = jnp.concatenate([src, zpad_i])
    dst_g = jnp.concatenate([dst, zpad_i])
    dst_s = jnp.concatenate([dst, jnp.full((pad,), N, jnp.int32)])
    ef_p = jnp.concatenate([ef, jnp.zeros((pad, ED), ef.dtype)])

    h = _tc_mm_bias(nf, W_in, b_in, br=2000)
    for l in range(msg_w1.shape[0]):
        tables = _tc_tables(h, msg_w1[l][0:2 * HD], br=2000)
        pre = _sc_gather(tables[0], tables[1], src_g, dst_g)
        m = _tc_edge_mlp(pre, ef_p, msg_w1[l][2 * HD:], msg_b1[l],
                         msg_w2[l], msg_b2[l], be=4096)
        agg = _sc_scatter(m, dst_s)
        h = _tc_update(h, agg, upd_w1[l], upd_b1[l], upd_w2[l], upd_b2[l],
                       br=2000)
    out = _tc_mm_bias(h, W_out, b_out, br=2000)
    return out[None]


# speedup vs baseline: 10.3522x; 1.4043x over previous
import functools

import jax
import jax.numpy as jnp
from jax import lax
from jax.experimental import pallas as pl
from jax.experimental.pallas import tpu as pltpu
from jax.experimental.pallas import tpu_sc as plsc

N = 50000
E = 800000
ND = 128
HD = 64
ED = 16

NC = 2
NS = 16
CG = 128
E_PAD = 32 * 196 * CG
E_H = E_PAD // 2
CH = CG // 2
EW = E_PAD // (NC * NS)
ET = E_PAD // NS
NROWS_SP = 50016

_SC_MESH = plsc.VectorSubcoreMesh(core_axis_name="c", subcore_axis_name="s")
_SC_PARAMS = pltpu.CompilerParams(use_tc_tiling_on_sc=False)


def _mm_bias_body(x_ref, w_ref, b_ref, o_ref):
    o_ref[...] = jnp.dot(x_ref[...], w_ref[...],
                         preferred_element_type=jnp.float32) + b_ref[...]


def _tc_mm_bias(x, w, b, br):
    r, k = x.shape
    c = w.shape[1]
    return pl.pallas_call(
        _mm_bias_body,
        grid=(r // br,),
        in_specs=[pl.BlockSpec((br, k), lambda i: (i, 0)),
                  pl.BlockSpec((k, c), lambda i: (0, 0)),
                  pl.BlockSpec((1, c), lambda i: (0, 0))],
        out_specs=pl.BlockSpec((br, c), lambda i: (i, 0)),
        out_shape=jax.ShapeDtypeStruct((r, c), jnp.float32),
    )(x, w, b.reshape(1, c))


def _tables_body(h_ref, w_ref, o_ref):
    h = h_ref[...]
    o_ref[0] = jnp.dot(h, w_ref[0:HD], preferred_element_type=jnp.float32)
    o_ref[1] = jnp.dot(h, w_ref[HD:2 * HD], preferred_element_type=jnp.float32)


def _tc_tables(h, w12, br):
    return pl.pallas_call(
        _tables_body,
        grid=(N // br,),
        in_specs=[pl.BlockSpec((br, HD), lambda i: (i, 0)),
                  pl.BlockSpec((2 * HD, HD), lambda i: (0, 0))],
        out_specs=pl.BlockSpec((2, br, HD), lambda i: (0, i, 0)),
        out_shape=jax.ShapeDtypeStruct((2, N, HD), jnp.float32),
    )(h, w12)


def _edge_mlp_body(pre_ref, ef_ref, wc_ref, b1_ref, w2_ref, b2_ref, o_ref):
    mi = pre_ref[...] + jnp.dot(ef_ref[...], wc_ref[...],
                                preferred_element_type=jnp.float32) + b1_ref[...]
    o_ref[...] = jnp.dot(jax.nn.gelu(mi), w2_ref[...],
                         preferred_element_type=jnp.float32) + b2_ref[...]


def _tc_edge_mlp(pre2, ef2, wc2, b1d, w2d, b2d, be):
    return pl.pallas_call(
        _edge_mlp_body,
        grid=(E_H // be,),
        in_specs=[pl.BlockSpec((be, 2 * HD), lambda i: (i, 0)),
                  pl.BlockSpec((be, 2 * ED), lambda i: (i, 0)),
                  pl.BlockSpec((2 * ED, 2 * HD), lambda i: (0, 0)),
                  pl.BlockSpec((1, 2 * HD), lambda i: (0, 0)),
                  pl.BlockSpec((2 * HD, 2 * HD), lambda i: (0, 0)),
                  pl.BlockSpec((1, 2 * HD), lambda i: (0, 0))],
        out_specs=pl.BlockSpec((be, 2 * HD), lambda i: (i, 0)),
        out_shape=jax.ShapeDtypeStruct((E_H, 2 * HD), jnp.float32),
    )(pre2, ef2, wc2, b1d.reshape(1, 2 * HD), w2d, b2d.reshape(1, 2 * HD))


def _update_body(h_ref, a_ref, w1_ref, b1_ref, w2_ref, b2_ref, o_ref):
    h = h_ref[...]
    ui = (jnp.dot(h, w1_ref[0:HD], preferred_element_type=jnp.float32)
          + jnp.dot(a_ref[...], w1_ref[HD:2 * HD],
                    preferred_element_type=jnp.float32)
          + b1_ref[...])
    o_ref[...] = h + jnp.dot(jax.nn.gelu(ui), w2_ref[...],
                             preferred_element_type=jnp.float32) + b2_ref[...]


def _tc_update(h, agg, w1, b1, w2, b2, br):
    return pl.pallas_call(
        _update_body,
        grid=(N // br,),
        in_specs=[pl.BlockSpec((br, HD), lambda i: (i, 0)),
                  pl.BlockSpec((br, HD), lambda i: (i, 0)),
                  pl.BlockSpec((2 * HD, HD), lambda i: (0, 0)),
                  pl.BlockSpec((1, HD), lambda i: (0, 0)),
                  pl.BlockSpec((HD, HD), lambda i: (0, 0)),
                  pl.BlockSpec((1, HD), lambda i: (0, 0))],
        out_specs=pl.BlockSpec((br, HD), lambda i: (i, 0)),
        out_shape=jax.ShapeDtypeStruct((N, HD), jnp.float32),
    )(h, agg, w1, b1.reshape(1, HD), w2, b2.reshape(1, HD))


def _gather_body(ts_ref, td_ref, se_ref, so_ref, de_ref, do_ref, pre_ref,
                 ise, iso, ide, ido, bse, bso, bde, bdo, sems):
    cid = lax.axis_index("c")
    sid = lax.axis_index("s")
    base = (sid * NC + cid) * (EW // 2)
    nchunks = EW // CG

    idxs = (ise, iso, ide, ido)
    bufs = (bse, bso, bde, bdo)
    srcs = (se_ref, so_ref, de_ref, do_ref)
    tabs = (ts_ref, ts_ref, td_ref, td_ref)

    def issue(slot, g):
        p0 = base + g * CH
        for k in range(4):
            pltpu.sync_copy(srcs[k].at[pl.ds(p0, CH)], idxs[k][slot])
        for k in range(4):
            pltpu.async_copy(tabs[k].at[idxs[k][slot]], bufs[k][slot],
                             sems[slot])

    def finish(slot, g):
        for k in range(4):
            pltpu.make_async_copy(tabs[k].at[idxs[k][slot]], bufs[k][slot],
                                  sems[slot]).wait()

        def addrow(i, c2):
            for k in range(2):
                for c4 in range(HD // 16):
                    sl = pl.ds(c4 * 16, 16)
                    bufs[k][slot][i, sl] = (bufs[k][slot][i, sl]
                                            + bufs[k + 2][slot][i, sl])
            return c2

        lax.fori_loop(0, CH, addrow, 0)
        r0 = base + g * CH
        pltpu.sync_copy(bse[slot], pre_ref.at[pl.ds(r0, CH), pl.ds(0, HD)])
        pltpu.sync_copy(bso[slot], pre_ref.at[pl.ds(r0, CH), pl.ds(HD, HD)])

    issue(0, 0)

    def pair(g2, carry):
        g = g2 * 2
        issue(1, g + 1)
        finish(0, g)

        @pl.when(g + 2 < nchunks)
        def _():
            issue(0, g + 2)

        finish(1, g + 1)
        return carry

    lax.fori_loop(0, nchunks // 2, pair, 0)


def _dbuf(shape, dtype):
    return [pltpu.VMEM(shape, dtype), pltpu.VMEM(shape, dtype)]


_sc_gather = pl.kernel(
    _gather_body,
    out_type=jax.ShapeDtypeStruct((E_H, 2 * HD), jnp.float32),
    mesh=_SC_MESH,
    scratch_types=[
        _dbuf((CH,), jnp.int32),
        _dbuf((CH,), jnp.int32),
        _dbuf((CH,), jnp.int32),
        _dbuf((CH,), jnp.int32),
        _dbuf((CH, HD), jnp.float32),
        _dbuf((CH, HD), jnp.float32),
        _dbuf((CH, HD), jnp.float32),
        _dbuf((CH, HD), jnp.float32),
        [pltpu.SemaphoreType.DMA, pltpu.SemaphoreType.DMA],
    ],
    compiler_params=_SC_PARAMS,
)


def _scatter_body(m_ref, dse_ref, dso_ref, agg_ref, ie, io, me, mo, ob, aggs,
                  sems):
    cid = lax.axis_index("c")
    sid = lax.axis_index("s")
    col0 = cid * (HD // NC)
    hw = HD // NC

    def zrow(i, carry):
        ob[i, pl.ds(0, 16)] = jnp.zeros((16,), jnp.float32)
        ob[i, pl.ds(16, 16)] = jnp.zeros((16,), jnp.float32)
        return carry

    lax.fori_loop(0, 125, zrow, 0)

    def zcopy(j, carry):
        pltpu.sync_copy(ob, aggs.at[pl.ds(sid * 3126 + j * 125, 125)])
        return carry

    lax.fori_loop(0, 25, zcopy, 0)
    pltpu.sync_copy(ob.at[pl.ds(0, 1)], aggs.at[pl.ds(sid * 3126 + 3125, 1)])
    plsc.subcore_barrier()

    base = sid * (ET // 2)
    nchunks = ET // CG

    def issue(slot, g):
        p0 = base + g * CH
        pltpu.sync_copy(dse_ref.at[pl.ds(p0, CH)], ie[slot])
        pltpu.sync_copy(dso_ref.at[pl.ds(p0, CH)], io[slot])
        pltpu.async_copy(m_ref.at[pl.ds(p0, CH), pl.ds(col0, hw)],
                         me[slot], sems[slot])
        pltpu.async_copy(m_ref.at[pl.ds(p0, CH), pl.ds(HD + col0, hw)],
                         mo[slot], sems[slot])

    def finish(slot, g):
        p0 = base + g * CH
        pltpu.make_async_copy(m_ref.at[pl.ds(p0, CH), pl.ds(col0, hw)],
                              me[slot], sems[slot]).wait()
        pltpu.make_async_copy(m_ref.at[pl.ds(p0, CH), pl.ds(HD + col0, hw)],
                              mo[slot], sems[slot]).wait()
        pltpu.sync_copy(me[slot], aggs.at[ie[slot]], add=True)
        pltpu.sync_copy(mo[slot], aggs.at[io[slot]], add=True)

    issue(0, 0)

    def pair(g2, carry):
        g = g2 * 2
        issue(1, g + 1)
        finish(0, g)

        @pl.when(g + 2 < nchunks)
        def _():
            issue(0, g + 2)

        finish(1, g + 1)
        return carry

    lax.fori_loop(0, nchunks // 2, pair, 0)
    plsc.subcore_barrier()

    def wout(k, carry):
        r0 = sid * 3125 + k * 125
        pltpu.sync_copy(aggs.at[pl.ds(r0, 125)], ob)
        pltpu.sync_copy(ob, agg_ref.at[pl.ds(r0, 125), pl.ds(col0, hw)])
        return carry

    lax.fori_loop(0, 25, wout, 0)


_sc_scatter = pl.kernel(
    _scatter_body,
    out_type=jax.ShapeDtypeStruct((N, HD), jnp.float32),
    mesh=_SC_MESH,
    scratch_types=[
        _dbuf((CH,), jnp.int32),
        _dbuf((CH,), jnp.int32),
        _dbuf((CH, HD // NC), jnp.float32),
        _dbuf((CH, HD // NC), jnp.float32),
        pltpu.VMEM((125, HD // NC), jnp.float32),
        pltpu.VMEM_SHARED((NROWS_SP, HD // NC), jnp.float32),
        [pltpu.SemaphoreType.DMA, pltpu.SemaphoreType.DMA],
    ],
    compiler_params=_SC_PARAMS,
)


def kernel(node_features, edge_indices, edge_features, W_in, b_in,
           msg_w1, msg_b1, msg_w2, msg_b2,
           upd_w1, upd_b1, upd_w2, upd_b2, W_out, b_out):
    nf = node_features[0]
    src = edge_indices[0, :, 0].astype(jnp.int32)
    dst = edge_indices[0, :, 1].astype(jnp.int32)
    ef = edge_features[0]

    pad = E_PAD - E
    zpad_i = jnp.zeros((pad,), jnp.int32)
    src_g = jnp.concatenate([src, zpad_i])
    dst_g = jnp.concatenate([dst, zpad_i])
    dst_s = jnp.concatenate([dst, jnp.full((pad,), N, jnp.int32)])
    ef2 = jnp.concatenate([ef, jnp.zeros((pad, ED), ef.dtype)]
                          ).reshape(E_H, 2 * ED)
    se, so = src_g[0::2], src_g[1::2]
    de, do = dst_g[0::2], dst_g[1::2]
    dse, dso = dst_s[0::2], dst_s[1::2]

    z = jnp.zeros((HD, HD), jnp.float32)
    ze = jnp.zeros((ED, HD), jnp.float32)

    h = _tc_mm_bias(nf, W_in, b_in, br=2000)
    for l in range(msg_w1.shape[0]):
        wc = msg_w1[l][2 * HD:]
        wc2 = jnp.concatenate(
            [jnp.concatenate([wc, ze], 1), jnp.concatenate([ze, wc], 1)], 0)
        w2d = jnp.concatenate(
            [jnp.concatenate([msg_w2[l], z], 1),
             jnp.concatenate([z, msg_w2[l]], 1)], 0)
        b1d = jnp.concatenate([msg_b1[l], msg_b1[l]])
        b2d = jnp.concatenate([msg_b2[l], msg_b2[l]])

        tables = _tc_tables(h, msg_w1[l][0:2 * HD], br=2000)
        pre2 = _sc_gather(tables[0], tables[1], se, so, de, do)
        m2 = _tc_edge_mlp(pre2, ef2, wc2, b1d, w2d, b2d, be=2048)
        agg = _sc_scatter(m2, dse, dso)
        h = _tc_update(h, agg, upd_w1[l], upd_b1[l], upd_w2[l], upd_b2[l],
                       br=2000)
    out = _tc_mm_bias(h, W_out, b_out, br=2000)
    return out[None]
